# Initial kernel scaffold; baseline (speedup 1.0000x reference)
import jax
import jax.numpy as jnp
from jax.experimental import pallas as pl


def kernel(input, T0, T1, T2, T3, T4):
    raise NotImplementedError("write your pallas kernel here")


# speedup vs baseline: 4.2814x; 4.2814x over previous
import functools

import jax
import jax.numpy as jnp
from jax import lax
from jax.experimental import pallas as pl
from jax.experimental.pallas import tpu as pltpu
from jax.experimental.pallas import tpu_sc as plsc

B, L, F, D = 4096, 50, 5, 64
N = B * L

NC, NS = 2, 16
NW = NC * NS
RPW = N // NW
CH = 640
NB = RPW // CH


def _emb(idxT, t0, t1, t2, t3, t4):
    mesh = plsc.VectorSubcoreMesh(core_axis_name="c", subcore_axis_name="s")

    @functools.partial(
        pl.kernel,
        out_type=jax.ShapeDtypeStruct((N, F, D), jnp.float32),
        mesh=mesh,
        scratch_types=[
            pltpu.VMEM((CH,), jnp.int32),
            pltpu.VMEM((CH, 1, D), jnp.float32),
            pltpu.SemaphoreType.DMA,
        ],
        compiler_params=pltpu.CompilerParams(use_tc_tiling_on_sc=False),
    )
    def body(idx_hbm, T0, T1, T2, T3, T4, out_hbm, idx_v, rows_v, sem):
        tables = [T0, T1, T2, T3, T4]
        wid = lax.axis_index("s") * NC + lax.axis_index("c")
        wbase = wid * RPW
        for f in range(F):
            tab = tables[f]

            def step(i, _, tab=tab, f=f):
                n0 = pl.multiple_of(wbase + i * CH, 8)
                pltpu.sync_copy(idx_hbm.at[pl.ds(f * N + n0, CH)], idx_v)
                pltpu.async_copy(tab.at[idx_v], rows_v.at[:, 0], sem).wait()
                pltpu.sync_copy(rows_v, out_hbm.at[pl.ds(n0, CH), pl.ds(f, 1)])
                return ()

            lax.fori_loop(0, NB, step, ())

    return body(idxT, t0, t1, t2, t3, t4)


def kernel(input, T0, T1, T2, T3, T4):
    idxT = input.reshape(N, F).T.reshape(-1)
    out = _emb(idxT, T0, T1, T2, T3, T4)
    return out.reshape(B, L, F, D)


# speedup vs baseline: 4.4432x; 1.0378x over previous
import functools

import jax
import jax.numpy as jnp
from jax import lax
from jax.experimental import pallas as pl
from jax.experimental.pallas import tpu as pltpu
from jax.experimental.pallas import tpu_sc as plsc

B, L, F, D = 4096, 50, 5, 64
N = B * L

NC, NS = 2, 16
NW = NC * NS
RPW = N // NW
CH = 640
NB = RPW // CH


def _emb(idxT, t0, t1, t2, t3, t4):
    mesh = plsc.VectorSubcoreMesh(core_axis_name="c", subcore_axis_name="s")

    @functools.partial(
        pl.kernel,
        out_type=jax.ShapeDtypeStruct((N, F, D), jnp.float32),
        mesh=mesh,
        scratch_types=[
            pltpu.VMEM((F * RPW,), jnp.int32),
            pltpu.VMEM((CH, 1, D), jnp.float32),
            pltpu.VMEM((CH, 1, D), jnp.float32),
            pltpu.SemaphoreType.DMA,
            pltpu.SemaphoreType.DMA,
            pltpu.SemaphoreType.DMA,
            pltpu.SemaphoreType.DMA,
        ],
        compiler_params=pltpu.CompilerParams(use_tc_tiling_on_sc=False),
    )
    def body(idx_hbm, T0, T1, T2, T3, T4, out_hbm,
             idx_all, rows0, rows1, gs0, gs1, ss0, ss1):
        tables = [T0, T1, T2, T3, T4]
        bufs, gsem, ssem = [rows0, rows1], [gs0, gs1], [ss0, ss1]
        wid = lax.axis_index("s") * NC + lax.axis_index("c")
        wbase = pl.multiple_of(wid * RPW, 8)

        for f in range(F):
            pltpu.sync_copy(idx_hbm.at[pl.ds(f * N + wbase, RPW)],
                            idx_all.at[pl.ds(f * RPW, RPW)])

        T = F * NB
        gath, scat = [None, None], [None, None]

        def start_gather(t):
            f, i, b = t // NB, t % NB, t % 2
            idx = idx_all.at[pl.ds((f * NB + i) * CH, CH)]
            gath[b] = pltpu.async_copy(tables[f].at[idx], bufs[b].at[:, 0],
                                       gsem[b])

        def start_scatter(t):
            f, i, b = t // NB, t % NB, t % 2
            n0 = pl.multiple_of(wbase + i * CH, 8)
            scat[b] = pltpu.async_copy(
                bufs[b], out_hbm.at[pl.ds(n0, CH), pl.ds(f, 1)], ssem[b])

        start_gather(0)
        for t in range(T):
            b, nb = t % 2, (t + 1) % 2
            if t + 1 < T:
                if scat[nb] is not None:
                    scat[nb].wait()
                start_gather(t + 1)
            gath[b].wait()
            start_scatter(t)
        scat[0].wait()
        scat[1].wait()

    return body(idxT, t0, t1, t2, t3, t4)


def kernel(input, T0, T1, T2, T3, T4):
    idxT = input.reshape(N, F).T.reshape(-1)
    out = _emb(idxT, T0, T1, T2, T3, T4)
    return out.reshape(B, L, F, D)
